# baseline (speedup 1.0000x reference)
import jax
import jax.numpy as jnp
import numpy as np
from jax import lax
from jax.experimental import pallas as pl
from jax.experimental.pallas import tpu as pltpu
from jax.experimental.pallas import tpu_sc as plsc

B, N, D, K = 8, 8192, 512, 2048
NC, NS = 2, 16
RPC = B // NC
WPR = NS // RPC
IPW = K // WPR
CHUNK = 64
RADIX_BITS = 8
RADIX = 1 << RADIX_BITS
PASSES = 32 // RADIX_BITS
SIGN = np.uint32(0x80000000)
NOSIGN = np.uint32(0x7FFFFFFF)


def _sc_body(w_hbm, f_hbm, c_hbm, fout_hbm, cout_hbm, wout_hbm,
             wrow, kk0, id0, kk1, id1, hist, off, wtop, crow, cbuf,
             idx_v, gidx_v, wv, fbuf, idx_sh, w_sh, sem):
  c = lax.axis_index("c")
  s = lax.axis_index("s")
  iota = lax.iota(jnp.int32, 16)
  ones = jnp.ones((16,), jnp.int32)

  @pl.when(s < RPC)
  def _sort_phase():
    r = c * RPC + s
    pltpu.sync_copy(w_hbm.at[pl.ds(r * N, N)], wrow)

    def build(i, _):
      sl = pl.ds(i * 16, 16)
      w16 = wrow[sl]
      u = plsc.bitcast(w16, jnp.uint32)
      neg = plsc.bitcast(w16, jnp.int32) < 0
      kk = jnp.where(neg, u, (~u) & NOSIGN)
      kk0[sl] = plsc.bitcast(kk, jnp.int32)
      id0[sl] = iota + i * 16
      return 0
    lax.fori_loop(0, N // 16, build, 0, unroll=4)

    for p in range(PASSES):
      kin, iin, kout, iout = ((kk0, id0, kk1, id1) if p % 2 == 0
                              else (kk1, id1, kk0, id0))
      shift = p * RADIX_BITS
      mask = jnp.uint32(RADIX - 1)

      def zero(i, _):
        hist[pl.ds(i * 16, 16)] = jnp.zeros((16,), jnp.int32)
        return 0
      lax.fori_loop(0, RADIX // 16, zero, 0, unroll=4)

      def histo(i, _, kin=kin, shift=shift, mask=mask):
        kk = plsc.bitcast(kin[pl.ds(i * 16, 16)], jnp.uint32)
        d = ((kk >> shift) & mask).astype(jnp.int32)
        plsc.addupdate_scatter(hist, [d], ones)
        return 0
      lax.fori_loop(0, N // 16, histo, 0, unroll=4)

      def scan(i, carry):
        sl = pl.ds(i * 16, 16)
        h = hist[sl]
        cs = plsc.cumsum(h)
        off[sl] = cs - h + carry
        return carry + jnp.sum(h)
      lax.fori_loop(0, RADIX // 16, scan, jnp.int32(0))

      def perm(i, _, kin=kin, iin=iin, kout=kout, iout=iout,
               shift=shift, mask=mask):
        sl = pl.ds(i * 16, 16)
        kki = kin[sl]
        idi = iin[sl]
        kk = plsc.bitcast(kki, jnp.uint32)
        d = ((kk >> shift) & mask).astype(jnp.int32)
        basev = plsc.load_gather(off, [d])
        cnt, _ = plsc.scan_count(d)
        pos = basev + cnt - 1
        plsc.store_scatter(kout, [pos], kki)
        plsc.store_scatter(iout, [pos], idi)
        plsc.addupdate_scatter(off, [d], ones)
        return 0
      lax.fori_loop(0, N // 16, perm, 0, unroll=2)

    kfin, ifin = (kk0, id0) if PASSES % 2 == 0 else (kk1, id1)

    def unkey(i, _):
      sl = pl.ds(i * 16, 16)
      kk = plsc.bitcast(kfin[sl], jnp.uint32)
      negk = kk >= SIGN
      u = jnp.where(negk, kk, (~kk) & NOSIGN)
      wtop[sl] = plsc.bitcast(u, jnp.float32)
      return 0
    lax.fori_loop(0, K // 16, unkey, 0, unroll=4)
    pltpu.sync_copy(wtop, wout_hbm.at[pl.ds(r * K, K)])

    for k3 in range(3):
      pltpu.sync_copy(c_hbm.at[pl.ds((k3 * B + r) * N, N)],
                      crow.at[pl.ds(k3 * N, N)])

    def cgather(i, _):
      j16 = iota + i * 16
      idx16 = ifin[pl.ds(i * 16, 16)]
      for k3 in range(3):
        vals = plsc.load_gather(crow, [idx16 + (k3 * N)])
        plsc.store_scatter(cbuf, [j16 * 3 + k3], vals)
      return 0
    lax.fori_loop(0, K // 16, cgather, 0, unroll=2)
    pltpu.sync_copy(cbuf, cout_hbm.at[pl.ds(r * K * 3, K * 3)])

    pltpu.sync_copy(ifin.at[pl.ds(0, K)], idx_sh.at[pl.ds(s * K, K)])
    pltpu.sync_copy(wtop, w_sh.at[pl.ds(s * K, K)])

  plsc.subcore_barrier()

  lr = s // WPR
  q = s - lr * WPR
  r = c * RPC + lr
  out_base = r * K + q * IPW
  pltpu.sync_copy(idx_sh.at[pl.ds(lr * K + q * IPW, IPW)], idx_v)
  pltpu.sync_copy(w_sh.at[pl.ds(lr * K + q * IPW, IPW)], wv)

  def glob(i, _):
    sl = pl.ds(i * 16, 16)
    gidx_v[sl] = idx_v[sl] + r * N
    return 0
  lax.fori_loop(0, IPW // 16, glob, 0, unroll=4)

  for t in range(IPW // CHUNK):
    pltpu.async_copy(
        f_hbm.at[gidx_v.at[pl.ds(t * CHUNK, CHUNK)]], fbuf, sem).wait()

    def scale(i, _, t=t):
      j = i >> 5
      m = i - (j << 5)
      rows = jnp.full((16,), 0, jnp.int32) + j
      cols = iota + m * 16
      wj = plsc.load_gather(wv, [jnp.full((16,), t * CHUNK, jnp.int32) + j])
      v = plsc.load_gather(fbuf, [rows, cols])
      plsc.store_scatter(fbuf, [rows, cols], v * wj)
      return 0
    lax.fori_loop(0, CHUNK * (D // 16), scale, 0, unroll=8)

    pltpu.sync_copy(fbuf, fout_hbm.at[pl.ds(out_base + t * CHUNK, CHUNK)])


def _get_kernel():
  mesh = plsc.VectorSubcoreMesh(core_axis_name="c", subcore_axis_name="s",
                                num_cores=NC, num_subcores=NS)
  return pl.kernel(
      _sc_body,
      out_type=(jax.ShapeDtypeStruct((B * K, D), jnp.float32),
                jax.ShapeDtypeStruct((B * K * 3,), jnp.float32),
                jax.ShapeDtypeStruct((B * K,), jnp.float32)),
      mesh=mesh,
      compiler_params=pltpu.CompilerParams(needs_layout_passes=False),
      scratch_types=[
          pltpu.VMEM((N,), jnp.float32),
          pltpu.VMEM((N,), jnp.int32),
          pltpu.VMEM((N,), jnp.int32),
          pltpu.VMEM((N,), jnp.int32),
          pltpu.VMEM((N,), jnp.int32),
          pltpu.VMEM((RADIX,), jnp.int32),
          pltpu.VMEM((RADIX,), jnp.int32),
          pltpu.VMEM((K,), jnp.float32),
          pltpu.VMEM((3 * N,), jnp.float32),
          pltpu.VMEM((K * 3,), jnp.float32),
          pltpu.VMEM((IPW,), jnp.int32),
          pltpu.VMEM((IPW,), jnp.int32),
          pltpu.VMEM((IPW,), jnp.float32),
          pltpu.VMEM((CHUNK, D), jnp.float32),
          pltpu.VMEM_SHARED((RPC * K,), jnp.int32),
          pltpu.VMEM_SHARED((RPC * K,), jnp.float32),
          pltpu.SemaphoreType.DMA,
      ],
  )


def kernel(node_weights, node_feats, coords):
  w1 = node_weights.reshape(B * N)
  f2 = node_feats.reshape(B * N, D)
  ct = jnp.transpose(coords, (2, 0, 1)).reshape(3 * B * N)
  fout, cout, wout = _get_kernel()(w1, f2, ct)
  return (fout.reshape(B, K, D), cout.reshape(B, K, 3),
          wout.reshape(B, K))
